# Initial kernel scaffold; baseline (speedup 1.0000x reference)
import jax
import jax.numpy as jnp
from jax.experimental import pallas as pl


def kernel(input_ids):
    raise NotImplementedError("write your pallas kernel here")


# speedup vs baseline: 2.1592x; 2.1592x over previous
import dataclasses
import functools

import jax
import jax.numpy as jnp
from jax import lax
from jax.experimental import pallas as pl
from jax.experimental.pallas import tpu as pltpu
from jax.experimental.pallas import tpu_sc as plsc

PAD = 0
B = 1024
L = 200
LANES = 16
LP = 208
V = 30522
VP = 30528
NC = 2
NS = 16
NW = NC * NS
RPW = B // NW
NCHUNK = LP // LANES

_mesh = plsc.VectorSubcoreMesh(core_axis_name="c", subcore_axis_name="s")

_cp = pltpu.CompilerParams()
if "needs_layout_passes" in pltpu.CompilerParams.__dataclass_fields__:
    _cp = dataclasses.replace(_cp, needs_layout_passes=False)
if "use_tc_tiling_on_sc" in pltpu.CompilerParams.__dataclass_fields__:
    _cp = dataclasses.replace(_cp, use_tc_tiling_on_sc=False)


@functools.partial(
    pl.kernel,
    out_type=jax.ShapeDtypeStruct((B, V), jnp.float32),
    mesh=_mesh,
    scratch_types=[
        pltpu.VMEM((RPW, LP), jnp.int32),
        pltpu.VMEM((VP,), jnp.float32),
        pltpu.SemaphoreType.DMA,
    ],
    compiler_params=_cp,
)
def _bow(ids_hbm, out_hbm, ids_v, hist_v, sem):
    wid = lax.axis_index("s") * NC + lax.axis_index("c")
    base = wid * RPW

    pltpu.sync_copy(ids_hbm.at[pl.ds(base, RPW)], ids_v)

    zeros_f = jnp.zeros((LANES,), jnp.float32)
    ones_f = jnp.ones((LANES,), jnp.float32)

    @pl.loop(0, VP, step=LANES)
    def _(i):
        hist_v[pl.ds(i, LANES)] = zeros_f

    @pl.loop(0, RPW)
    def _(r):
        carry = jnp.int32(0)
        for c in range(NCHUNK):
            ids16 = ids_v[r, pl.ds(c * LANES, LANES)]
            ip = (ids16 == PAD).astype(jnp.int32)
            cum = plsc.cumsum(ip)
            valid = (cum + carry) == 0
            cnt, last = plsc.scan_count(ids16, mask=valid)
            plsc.addupdate_scatter(
                hist_v, [ids16], cnt.astype(jnp.float32), mask=last & valid
            )
            carry = carry + jnp.sum(ip)
        pltpu.sync_copy(hist_v.at[pl.ds(0, V)], out_hbm.at[base + r])
        for c in range(NCHUNK):
            ids16 = ids_v[r, pl.ds(c * LANES, LANES)]
            plsc.store_scatter(hist_v, [ids16], zeros_f)


def kernel(input_ids):
    ids = jnp.pad(input_ids, ((0, 0), (0, LP - L)))
    return _bow(ids)
